# Initial kernel scaffold; baseline (speedup 1.0000x reference)
import jax
import jax.numpy as jnp
from jax.experimental import pallas as pl


def kernel(x, edge_index, W1, b1, W2, b2):
    raise NotImplementedError("write your pallas kernel here")


# speedup vs baseline: 144.2640x; 144.2640x over previous
import jax
import jax.numpy as jnp
from jax import lax
from jax.experimental import pallas as pl
from jax.experimental.pallas import tpu as pltpu
from jax.experimental.pallas import tpu_sc as plsc

N = 10000
D = 128
H = 64
K = 10
ALPHA = 0.1
E = 320000

NS = 16
L = 16
NODE_W = 640
N_PAD = NS * NODE_W
B = 128
CPT = 160
E_PAD = NS * CPT * B
NCH = E_PAD // B


def _mlp_body(x_ref, w1_ref, b1_ref, w2_ref, b2_ref, o_ref):
    t = jnp.dot(x_ref[...], w1_ref[...], preferred_element_type=jnp.float32)
    t = jnp.maximum(t + b1_ref[...][None, :], 0.0)
    w2col = w2_ref[...][:, 0]
    o_ref[...] = (jnp.sum(t * w2col[None, :], axis=1, keepdims=True)
                  + b2_ref[...][None, :])


def _sc_body(src_hbm, dst_hbm, h_hbm, z_hbm,
             u_sp, raw_sp, src_v, dst_v, vals_v,
             h_v, dinv_v, z_v, u_v, tmp_v, zero_v, ones_v,
             sem_ld, sem_g, sem_s):
    c = lax.axis_index("c")
    s = lax.axis_index("s")
    row0 = s * CPT
    nbase = s * NODE_W
    nsl = pl.ds(nbase, NODE_W)

    cp_a = pltpu.async_copy(src_hbm.at[pl.ds(row0, CPT)], src_v, sem_ld)
    cp_b = pltpu.async_copy(dst_hbm.at[pl.ds(row0, CPT)], dst_v, sem_ld)
    cp_c = pltpu.async_copy(h_hbm.at[nsl], h_v, sem_ld)
    cp_a.wait()
    cp_b.wait()
    cp_c.wait()

    for j in range(B // L):
        ones_v[pl.ds(j * L, L)] = jnp.full((L,), 1.0, jnp.float32)
    for j in range(NODE_W // L):
        zero_v[pl.ds(j * L, L)] = jnp.zeros((L,), jnp.float32)

    def fire_scatter_from(src_ref):
        def body(j, carry):
            pltpu.async_copy(src_ref.at[j] if src_ref is vals_v else src_ref,
                             raw_sp.at[dst_v.at[j]], sem_s, add=True)
            return carry
        return body

    def drain_scatter(j, carry):
        pltpu.make_async_copy(ones_v, raw_sp.at[dst_v.at[0]], sem_s).wait()
        return carry

    pltpu.sync_copy(zero_v, raw_sp.at[nsl])
    plsc.subcore_barrier()
    lax.fori_loop(0, CPT, fire_scatter_from(ones_v), 0)
    lax.fori_loop(0, CPT, drain_scatter, 0)
    plsc.subcore_barrier()

    pltpu.sync_copy(raw_sp.at[nsl], tmp_v)
    for j in range(NODE_W // L):
        sl = pl.ds(j * L, L)
        deg = tmp_v[sl] + 1.0
        bits = plsc.bitcast(deg, jnp.int32)
        y = plsc.bitcast(jnp.int32(0x5F3759DF) - (bits >> 1), jnp.float32)
        half = 0.5 * deg
        y = y * (1.5 - half * y * y)
        y = y * (1.5 - half * y * y)
        y = y * (1.5 - half * y * y)
        gidx = lax.iota(jnp.int32, L) + (nbase + j * L)
        y = jnp.where(gidx < N, y, 0.0)
        dinv_v[sl] = y
        hv = h_v[sl]
        z_v[sl] = hv
        u_v[sl] = y * hv

    def iter_body(k, carry):
        pltpu.sync_copy(zero_v, raw_sp.at[nsl])
        pltpu.sync_copy(u_v, u_sp.at[nsl])
        plsc.subcore_barrier()

        def fire_gather(j, cy):
            pltpu.async_copy(u_sp.at[src_v.at[j]], vals_v.at[j], sem_g)
            return cy
        lax.fori_loop(0, CPT, fire_gather, 0)

        def drain_gather(j, cy):
            pltpu.make_async_copy(u_sp.at[src_v.at[0]], vals_v.at[0],
                                  sem_g).wait()
            return cy
        lax.fori_loop(0, CPT, drain_gather, 0)

        lax.fori_loop(0, CPT, fire_scatter_from(vals_v), 0)
        lax.fori_loop(0, CPT, drain_scatter, 0)
        plsc.subcore_barrier()

        pltpu.sync_copy(raw_sp.at[nsl], tmp_v)
        for j in range(NODE_W // L):
            sl = pl.ds(j * L, L)
            dv = dinv_v[sl]
            znew = ((1.0 - ALPHA) * dv * (tmp_v[sl] + dv * z_v[sl])
                    + ALPHA * h_v[sl])
            z_v[sl] = znew
            u_v[sl] = dv * znew
        return carry
    lax.fori_loop(0, K, iter_body, 0)

    @pl.when(c == 0)
    def _():
        pltpu.sync_copy(z_v, z_hbm.at[nsl])


def kernel(x, edge_index, W1, b1, W2, b2):
    h = pl.pallas_call(
        _mlp_body,
        out_shape=jax.ShapeDtypeStruct((N, 1), jnp.float32),
    )(x, W1, b1, W2, b2)
    h_pad = jnp.pad(h[:, 0], (0, N_PAD - N))

    idx = edge_index.astype(jnp.int32)
    pad = E_PAD - E
    pidx = jnp.arange(pad, dtype=jnp.int32)
    src2 = jnp.concatenate([idx[0], pidx % N]).reshape(NCH, B)
    dst2 = jnp.concatenate([idx[1], N + pidx % (N_PAD - N)]).reshape(NCH, B)

    mesh = plsc.VectorSubcoreMesh(core_axis_name="c", subcore_axis_name="s")
    z_pad = pl.kernel(
        _sc_body,
        out_type=jax.ShapeDtypeStruct((N_PAD,), jnp.float32),
        mesh=mesh,
        compiler_params=pltpu.CompilerParams(needs_layout_passes=False),
        scratch_types=[
            pltpu.VMEM_SHARED((N_PAD,), jnp.float32),
            pltpu.VMEM_SHARED((N_PAD,), jnp.float32),
            pltpu.VMEM((CPT, B), jnp.int32),
            pltpu.VMEM((CPT, B), jnp.int32),
            pltpu.VMEM((CPT, B), jnp.float32),
            pltpu.VMEM((NODE_W,), jnp.float32),
            pltpu.VMEM((NODE_W,), jnp.float32),
            pltpu.VMEM((NODE_W,), jnp.float32),
            pltpu.VMEM((NODE_W,), jnp.float32),
            pltpu.VMEM((NODE_W,), jnp.float32),
            pltpu.VMEM((NODE_W,), jnp.float32),
            pltpu.VMEM((B,), jnp.float32),
            pltpu.SemaphoreType.DMA,
            pltpu.SemaphoreType.DMA,
            pltpu.SemaphoreType.DMA,
        ],
    )(src2, dst2, h_pad)
    return z_pad[:N, None]
